# Initial kernel scaffold; baseline (speedup 1.0000x reference)
import jax
import jax.numpy as jnp
from jax import lax
from jax.experimental import pallas as pl
from jax.experimental.pallas import tpu as pltpu
from jax.experimental.pallas import tpu_sc as plsc

N = 10000
E = 320000
F = 128
HID = 64
OUT = 64

NC = 2
NS = 16
NW = NC * NS

EC_W = 79
E_PAD = NW * EC_W * 128
EROWS = E_PAD // 128
EC_T = EROWS // NS

NP_T = 640
N_PAD = NS * NP_T

_MESH = plsc.VectorSubcoreMesh(
    core_axis_name="c", subcore_axis_name="s", num_cores=NC, num_subcores=NS)


def _rsqrt_masked(d):
    dn = jnp.maximum(d, jnp.float32(1e-35))
    i = plsc.bitcast(dn, jnp.int32)
    y = plsc.bitcast(jnp.int32(0x5F3759DF) - (i >> 1), jnp.float32)
    for _ in range(3):
        y = y * (jnp.float32(1.5) - jnp.float32(0.5) * dn * y * y)
    return jnp.where(d > jnp.float32(0.0), y, jnp.float32(0.0))


def _sc_degrees(rowm, colm, ewm, disA, deg_sh, row_v, col_v, ew_v, zb_v,
                dloc_v):
    cid = lax.axis_index("c")
    sid = lax.axis_index("s")

    for j in range(NP_T // 16):
        zb_v[pl.ds(16 * j, 16)] = jnp.zeros((16,), jnp.float32)
    pltpu.sync_copy(zb_v, deg_sh.at[pl.ds(sid * NP_T, NP_T)])
    plsc.subcore_barrier()

    base = sid * EC_T
    pltpu.sync_copy(rowm.at[pl.ds(base, EC_T)], row_v)
    pltpu.sync_copy(colm.at[pl.ds(base, EC_T)], col_v)
    pltpu.sync_copy(ewm.at[pl.ds(base, EC_T)], ew_v)

    @pl.when(cid == 0)
    def _():
        def zero_self(c, _):
            for j in range(8):
                sl = pl.ds(16 * j, 16)
                r = row_v[c, sl]
                cc = col_v[c, sl]
                w = ew_v[c, sl]
                ew_v[c, sl] = jnp.where(r == cc, jnp.float32(0.0), w)
            return 0

        lax.fori_loop(0, EC_T, zero_self, 0)

        def scat(c, _):
            pltpu.sync_copy(ew_v.at[c], deg_sh.at[row_v.at[c]], add=True)
            return 0

        lax.fori_loop(0, EC_T, scat, 0)

    @pl.when(cid == 1)
    def _():
        def scat(c, _):
            pltpu.sync_copy(ew_v.at[c], deg_sh.at[col_v.at[c]], add=True)
            return 0

        lax.fori_loop(0, EC_T, scat, 0)

    plsc.subcore_barrier()

    pltpu.sync_copy(deg_sh.at[pl.ds(sid * NP_T, NP_T)], dloc_v)
    for j in range(NP_T // 16):
        sl = pl.ds(16 * j, 16)
        dloc_v[sl] = _rsqrt_masked(dloc_v[sl])
    pltpu.sync_copy(dloc_v, disA.at[cid, pl.ds(sid * NP_T, NP_T)])


def _sc_edge_pass(table, rowm, colm, ewm, disA, accp,
                  acc_sh, dis_v, dis2_v, row_v, col_v, ew_v, sc_v, sa_v,
                  grows, zb_v):
    cid = lax.axis_index("c")
    sid = lax.axis_index("s")
    wid = sid * NC + cid

    pltpu.sync_copy(disA.at[0], dis_v)
    pltpu.sync_copy(disA.at[1], dis2_v)

    for i in range(16):
        for j in range(8):
            zb_v[i, pl.ds(16 * j, 16)] = jnp.zeros((16,), jnp.float32)

    def zacc(jj, _):
        pltpu.sync_copy(zb_v, acc_sh.at[pl.ds(sid * NP_T + 16 * jj, 16)])
        return 0

    lax.fori_loop(0, NP_T // 16, zacc, 0)
    plsc.subcore_barrier()

    ebase = wid * EC_W
    pltpu.sync_copy(rowm.at[pl.ds(ebase, EC_W)], row_v)
    pltpu.sync_copy(colm.at[pl.ds(ebase, EC_W)], col_v)
    pltpu.sync_copy(ewm.at[pl.ds(ebase, EC_W)], ew_v)

    def chunk(c, _):
        for j in range(8):
            sl = pl.ds(16 * j, 16)
            r = row_v[c, sl]
            cc = col_v[c, sl]
            w = ew_v[c, sl]
            dr = plsc.load_gather(dis_v, [r])
            d2r = plsc.load_gather(dis2_v, [r])
            wz = jnp.where(r == cc, jnp.float32(0.0), w)
            sc_v[sl] = wz * dr
            sa_v[sl] = w * d2r

        pltpu.sync_copy(table.at[row_v.at[c]], grows)

        def edge(e, _):
            scv = jnp.full((16,), sc_v[e], jnp.float32)
            sav = jnp.full((16,), sa_v[e], jnp.float32)
            for j in range(4):
                sl = pl.ds(16 * j, 16)
                grows[e, sl] = grows[e, sl] * scv
            for j in range(4, 8):
                sl = pl.ds(16 * j, 16)
                grows[e, sl] = grows[e, sl] * sav
            return 0

        lax.fori_loop(0, 128, edge, 0, unroll=2)

        pltpu.sync_copy(grows, acc_sh.at[col_v.at[c]], add=True)
        return 0

    lax.fori_loop(0, EC_W, chunk, 0)
    plsc.subcore_barrier()

    pltpu.sync_copy(acc_sh.at[pl.ds(sid * NP_T, NP_T)],
                    accp.at[cid, pl.ds(sid * NP_T, NP_T)])


def _tc_pre(h_ref, x_ref, Wh_ref, bh_ref, Wx_ref, bx_ref, Wi_ref,
            table_ref, xp_ref, hp_ref):
    h = h_ref[...]
    x = x_ref[...]
    dnT = (((1,), (1,)), ((), ()))
    hp = lax.dot_general(h, Wh_ref[...], dnT,
                         preferred_element_type=jnp.float32) + bh_ref[...][None, :]
    xp = lax.dot_general(x, Wx_ref[...], dnT,
                         preferred_element_type=jnp.float32) + bx_ref[...][None, :]
    msg = lax.dot_general(xp, Wi_ref[...], (((1,), (0,)), ((), ())),
                          preferred_element_type=jnp.float32)
    table_ref[...] = jnp.concatenate([hp, msg], axis=1)
    xp_ref[...] = xp
    hp_ref[...] = hp


def _tc_post(accp_ref, disA_ref, xp_ref, hp_ref, Wc0_ref, Wc1_ref, bc_ref,
             Wr_ref, ba_ref, Wl_ref, bl_ref, out_ref):
    acc = accp_ref[0] + accp_ref[1]
    dis = disA_ref[0]
    dis2 = disA_ref[1]
    tx1 = (-dis)[:, None] * acc[:, :HID]
    prop = dis2[:, None] * acc[:, HID:]
    dnT = (((1,), (1,)), ((), ()))
    o_cheb = (lax.dot_general(hp_ref[...], Wc0_ref[...], dnT,
                              preferred_element_type=jnp.float32)
              + lax.dot_general(tx1, Wc1_ref[...], dnT,
                                preferred_element_type=jnp.float32)
              + bc_ref[...][None, :])
    o_arma = prop + lax.dot_general(xp_ref[...], Wr_ref[...],
                                    (((1,), (0,)), ((), ())),
                                    preferred_element_type=jnp.float32)
    o_arma = o_arma + ba_ref[...][None, :]
    o_arma = jnp.maximum(o_arma, jnp.float32(0.0))
    slope = jnp.float32(0.01)
    o1 = jnp.where(o_cheb >= 0, o_cheb, slope * o_cheb)
    o2 = jnp.where(o_arma >= 0, o_arma, slope * o_arma)
    out_ref[...] = (lax.dot_general(o1 + o2, Wl_ref[...], dnT,
                                    preferred_element_type=jnp.float32)
                    + bl_ref[...][None, :])


def kernel(h, x, edge_weight, Wh, bh, Wx, bx, Wc0, Wc1, bc, Wi, Wr, ba, Wl,
           bl, edge_index):
    row = edge_index[0].astype(jnp.int32)
    col = edge_index[1].astype(jnp.int32)
    ew = edge_weight.astype(jnp.float32)

    npad = E_PAD - E
    pad_idx = jnp.arange(npad, dtype=jnp.int32) % N
    rowm = jnp.concatenate([row, pad_idx]).reshape(EROWS, 128)
    colm = jnp.concatenate([col, pad_idx]).reshape(EROWS, 128)
    ewm = jnp.concatenate([ew, jnp.zeros((npad,), jnp.float32)]
                          ).reshape(EROWS, 128)

    h_pad = jnp.zeros((N_PAD, h.shape[1]), jnp.float32).at[:N].set(h)
    x_pad = jnp.zeros((N_PAD, x.shape[1]), jnp.float32).at[:N].set(x)

    table, xp, hp = pl.pallas_call(
        _tc_pre,
        out_shape=(
            jax.ShapeDtypeStruct((N_PAD, F), jnp.float32),
            jax.ShapeDtypeStruct((N_PAD, HID), jnp.float32),
            jax.ShapeDtypeStruct((N_PAD, HID), jnp.float32),
        ),
    )(h_pad, x_pad, Wh, bh, Wx, bx, Wi)

    disA = pl.kernel(
        _sc_degrees,
        out_type=jax.ShapeDtypeStruct((NC, N_PAD), jnp.float32),
        mesh=_MESH,
        scratch_types=[
            pltpu.VMEM_SHARED((N_PAD,), jnp.float32),
            pltpu.VMEM((EC_T, 128), jnp.int32),
            pltpu.VMEM((EC_T, 128), jnp.int32),
            pltpu.VMEM((EC_T, 128), jnp.float32),
            pltpu.VMEM((NP_T,), jnp.float32),
            pltpu.VMEM((NP_T,), jnp.float32),
        ],
    )(rowm, colm, ewm)

    accp = pl.kernel(
        _sc_edge_pass,
        out_type=jax.ShapeDtypeStruct((NC, N_PAD, F), jnp.float32),
        mesh=_MESH,
        scratch_types=[
            pltpu.VMEM_SHARED((N_PAD, F), jnp.float32),
            pltpu.VMEM((N_PAD,), jnp.float32),
            pltpu.VMEM((N_PAD,), jnp.float32),
            pltpu.VMEM((EC_W, 128), jnp.int32),
            pltpu.VMEM((EC_W, 128), jnp.int32),
            pltpu.VMEM((EC_W, 128), jnp.float32),
            pltpu.VMEM((128,), jnp.float32),
            pltpu.VMEM((128,), jnp.float32),
            pltpu.VMEM((128, F), jnp.float32),
            pltpu.VMEM((16, F), jnp.float32),
        ],
    )(table, rowm, colm, ewm, disA)

    o3_pad = pl.pallas_call(
        _tc_post,
        out_shape=jax.ShapeDtypeStruct((N_PAD, OUT), jnp.float32),
    )(accp, disA, xp, hp, Wc0, Wc1, bc, Wr, ba, Wl, bl)

    return (x, o3_pad[:N])


# speedup vs baseline: 31.5607x; 31.5607x over previous
import jax
import jax.numpy as jnp
from jax import lax
from jax.experimental import pallas as pl
from jax.experimental.pallas import tpu as pltpu
from jax.experimental.pallas import tpu_sc as plsc

N = 10000
E = 320000
HID = 64
OUT = 64
F = 128

NC = 2
NS = 16
NW = NC * NS

EC_W = 80
E_PAD = NW * EC_W * 128
EROWS = E_PAD // 128
EC_T = EROWS // NS
EB_W = 16

NP_T = 640
N_PAD = NS * NP_T


def _sc_mesh():
    return plsc.VectorSubcoreMesh(
        core_axis_name="c", subcore_axis_name="s",
        num_cores=NC, num_subcores=NS)


def _rsqrt_masked(d):
    dn = jnp.maximum(d, jnp.float32(1e-35))
    i = lax.bitcast_convert_type(dn, jnp.int32)
    y = lax.bitcast_convert_type(jnp.int32(0x5F3759DF) - (i >> 1),
                                 jnp.float32)
    for _ in range(3):
        y = y * (jnp.float32(1.5) - jnp.float32(0.5) * dn * y * y)
    return jnp.where(d > jnp.float32(0.0), y, jnp.float32(0.0))


def _sc_degrees(rowm, colm, ewm, disA,
                deg_sh, row_v, col_v, ew_v, zb_v, dloc_v):
    cid = lax.axis_index("c")
    sid = lax.axis_index("s")

    for j in range(NP_T // 16):
        zb_v[pl.ds(16 * j, 16)] = jnp.zeros((16,), jnp.float32)
    pltpu.sync_copy(zb_v, deg_sh.at[pl.ds(sid * NP_T, NP_T)])
    plsc.subcore_barrier()

    base = sid * EC_T
    pltpu.sync_copy(rowm.at[pl.ds(base, EC_T)], row_v)
    pltpu.sync_copy(colm.at[pl.ds(base, EC_T)], col_v)
    pltpu.sync_copy(ewm.at[pl.ds(base, EC_T)], ew_v)

    @pl.when(cid == 0)
    def _():
        def zero_self(c, _):
            for j in range(8):
                sl = pl.ds(16 * j, 16)
                r = row_v[c, sl]
                cc = col_v[c, sl]
                w = ew_v[c, sl]
                ew_v[c, sl] = jnp.where(r == cc, jnp.float32(0.0), w)
            return 0

        lax.fori_loop(0, EC_T, zero_self, 0)

        def scat(c, _):
            pltpu.sync_copy(ew_v.at[c], deg_sh.at[row_v.at[c]], add=True)
            return 0

        lax.fori_loop(0, EC_T, scat, 0)

    @pl.when(cid == 1)
    def _():
        def scat(c, _):
            pltpu.sync_copy(ew_v.at[c], deg_sh.at[col_v.at[c]], add=True)
            return 0

        lax.fori_loop(0, EC_T, scat, 0)

    plsc.subcore_barrier()

    pltpu.sync_copy(deg_sh.at[pl.ds(sid * NP_T, NP_T)], dloc_v)
    for j in range(NP_T // 16):
        sl = pl.ds(16 * j, 16)
        dloc_v[sl] = _rsqrt_masked(dloc_v[sl])
    pltpu.sync_copy(dloc_v, disA.at[cid, pl.ds(sid * NP_T, NP_T)])


def _sc_edge_pass(hp, msg, rowm, colm, ewm, disA, accp, tbl,
                  acc_sh, disl_v, dis2l_v, hblk_v, mblk_v, tblk_v,
                  row_v, col_v, ew_v, grows_v, zb_v):
    cid = lax.axis_index("c")
    sid = lax.axis_index("s")
    wid = sid * NC + cid
    nbase = sid * NP_T

    pltpu.sync_copy(disA.at[0, pl.ds(nbase, NP_T)], disl_v)
    pltpu.sync_copy(disA.at[1, pl.ds(nbase, NP_T)], dis2l_v)

    for i in range(16):
        for j in range(8):
            zb_v[i, pl.ds(16 * j, 16)] = jnp.zeros((16,), jnp.float32)

    def zacc(jj, _):
        pltpu.sync_copy(zb_v, acc_sh.at[pl.ds(nbase + 16 * jj, 16)])
        return 0

    lax.fori_loop(0, NP_T // 16, zacc, 0)

    def tb(b, _):
        rbase = nbase + 16 * b
        pltpu.sync_copy(hp.at[pl.ds(rbase, 16)], hblk_v)
        pltpu.sync_copy(msg.at[pl.ds(rbase, 16)], mblk_v)
        d16 = disl_v[pl.ds(16 * b, 16)]
        d216 = dis2l_v[pl.ds(16 * b, 16)]
        for k in range(16):
            dk = jnp.full((16,), d16[k], jnp.float32)
            d2k = jnp.full((16,), d216[k], jnp.float32)
            for j in range(4):
                sl = pl.ds(16 * j, 16)
                sl2 = pl.ds(HID + 16 * j, 16)
                tblk_v[k, sl] = hblk_v[k, sl] * dk
                tblk_v[k, sl2] = mblk_v[k, sl] * d2k
        pltpu.sync_copy(tblk_v, tbl.at[pl.ds(rbase, 16)])
        return 0

    lax.fori_loop(0, NP_T // 16, tb, 0)
    plsc.subcore_barrier()

    ebase = wid * EC_W

    def block(b, _):
        pltpu.sync_copy(rowm.at[pl.ds(ebase + EB_W * b, EB_W)], row_v)
        pltpu.sync_copy(colm.at[pl.ds(ebase + EB_W * b, EB_W)], col_v)
        pltpu.sync_copy(ewm.at[pl.ds(ebase + EB_W * b, EB_W)], ew_v)

        def chunk(c, _):
            pltpu.sync_copy(tbl.at[row_v.at[c]], grows_v)

            def egroup(g, _):
                sl16 = pl.ds(16 * g, 16)
                r16 = row_v[c, sl16]
                c16 = col_v[c, sl16]
                w16 = ew_v[c, sl16]
                wz16 = jnp.where(r16 == c16, jnp.float32(0.0), w16)
                for k in range(16):
                    e = 16 * g + k
                    wzk = jnp.full((16,), wz16[k], jnp.float32)
                    wk = jnp.full((16,), w16[k], jnp.float32)
                    for j in range(4):
                        sl = pl.ds(16 * j, 16)
                        sl2 = pl.ds(HID + 16 * j, 16)
                        grows_v[e, sl] = grows_v[e, sl] * wzk
                        grows_v[e, sl2] = grows_v[e, sl2] * wk
                return 0

            lax.fori_loop(0, 8, egroup, 0)

            pltpu.sync_copy(grows_v, acc_sh.at[col_v.at[c]], add=True)
            return 0

        lax.fori_loop(0, EB_W, chunk, 0)
        return 0

    lax.fori_loop(0, EC_W // EB_W, block, 0)
    plsc.subcore_barrier()

    pltpu.sync_copy(acc_sh.at[pl.ds(nbase, NP_T)],
                    accp.at[cid, pl.ds(nbase, NP_T)])


def _tc_pre(h_ref, x_ref, Wh_ref, bh_ref, Wx_ref, bx_ref, Wi_ref,
            hp_ref, msg_ref, xp_ref):
    h = h_ref[...]
    x = x_ref[...]
    dnT = (((1,), (1,)), ((), ()))
    hp = lax.dot_general(h, Wh_ref[...], dnT,
                         preferred_element_type=jnp.float32) + bh_ref[...][None, :]
    xp = lax.dot_general(x, Wx_ref[...], dnT,
                         preferred_element_type=jnp.float32) + bx_ref[...][None, :]
    msg = lax.dot_general(xp, Wi_ref[...], (((1,), (0,)), ((), ())),
                          preferred_element_type=jnp.float32)
    hp_ref[...] = hp
    msg_ref[...] = msg
    xp_ref[...] = xp


def _tc_post(accp_ref, disA_ref, xp_ref, hp_ref, Wc0_ref, Wc1_ref,
             bc_ref, Wr_ref, ba_ref, Wl_ref, bl_ref, out_ref):
    dis = disA_ref[0]
    dis2 = disA_ref[1]
    acc = accp_ref[0] + accp_ref[1]
    tx1 = (-dis)[:, None] * acc[:, :HID]
    prop = dis2[:, None] * acc[:, HID:]
    dnT = (((1,), (1,)), ((), ()))
    o_cheb = (lax.dot_general(hp_ref[...], Wc0_ref[...], dnT,
                              preferred_element_type=jnp.float32)
              + lax.dot_general(tx1, Wc1_ref[...], dnT,
                                preferred_element_type=jnp.float32)
              + bc_ref[...][None, :])
    o_arma = prop + lax.dot_general(xp_ref[...], Wr_ref[...],
                                    (((1,), (0,)), ((), ())),
                                    preferred_element_type=jnp.float32)
    o_arma = o_arma + ba_ref[...][None, :]
    o_arma = jnp.maximum(o_arma, jnp.float32(0.0))
    slope = jnp.float32(0.01)
    o1 = jnp.where(o_cheb >= 0, o_cheb, slope * o_cheb)
    o2 = jnp.where(o_arma >= 0, o_arma, slope * o_arma)
    out_ref[...] = (lax.dot_general(o1 + o2, Wl_ref[...], dnT,
                                    preferred_element_type=jnp.float32)
                    + bl_ref[...][None, :])


def kernel(h, x, edge_weight, Wh, bh, Wx, bx, Wc0, Wc1, bc, Wi, Wr, ba, Wl,
           bl, edge_index):
    row = edge_index[0].astype(jnp.int32)
    col = edge_index[1].astype(jnp.int32)
    ew = edge_weight.astype(jnp.float32)

    npad = E_PAD - E
    pad_idx = jnp.arange(npad, dtype=jnp.int32) % N
    rowm = jnp.concatenate([row, pad_idx]).reshape(EROWS, 128)
    colm = jnp.concatenate([col, pad_idx]).reshape(EROWS, 128)
    ewm = jnp.concatenate([ew, jnp.zeros((npad,), jnp.float32)]
                          ).reshape(EROWS, 128)

    h_pad = jnp.zeros((N_PAD, h.shape[1]), jnp.float32).at[:N].set(h)
    x_pad = jnp.zeros((N_PAD, x.shape[1]), jnp.float32).at[:N].set(x)

    hp, msg, xp = pl.pallas_call(
        _tc_pre,
        out_shape=(
            jax.ShapeDtypeStruct((N_PAD, HID), jnp.float32),
            jax.ShapeDtypeStruct((N_PAD, HID), jnp.float32),
            jax.ShapeDtypeStruct((N_PAD, HID), jnp.float32),
        ),
    )(h_pad, x_pad, Wh, bh, Wx, bx, Wi)

    disA = pl.kernel(
        _sc_degrees,
        out_type=jax.ShapeDtypeStruct((NC, N_PAD), jnp.float32),
        mesh=_sc_mesh(),
        compiler_params=pltpu.CompilerParams(needs_layout_passes=False),
        scratch_types=[
            pltpu.VMEM_SHARED((N_PAD,), jnp.float32),
            pltpu.VMEM((EC_T, 128), jnp.int32),
            pltpu.VMEM((EC_T, 128), jnp.int32),
            pltpu.VMEM((EC_T, 128), jnp.float32),
            pltpu.VMEM((NP_T,), jnp.float32),
            pltpu.VMEM((NP_T,), jnp.float32),
        ],
    )(rowm, colm, ewm)

    accp, _tbl = pl.kernel(
        _sc_edge_pass,
        out_type=(
            jax.ShapeDtypeStruct((NC, N_PAD, F), jnp.float32),
            jax.ShapeDtypeStruct((N_PAD, F), jnp.float32),
        ),
        mesh=_sc_mesh(),
        compiler_params=pltpu.CompilerParams(needs_layout_passes=False),
        scratch_types=[
            pltpu.VMEM_SHARED((N_PAD, F), jnp.float32),
            pltpu.VMEM((NP_T,), jnp.float32),
            pltpu.VMEM((NP_T,), jnp.float32),
            pltpu.VMEM((16, HID), jnp.float32),
            pltpu.VMEM((16, HID), jnp.float32),
            pltpu.VMEM((16, F), jnp.float32),
            pltpu.VMEM((EB_W, 128), jnp.int32),
            pltpu.VMEM((EB_W, 128), jnp.int32),
            pltpu.VMEM((EB_W, 128), jnp.float32),
            pltpu.VMEM((128, F), jnp.float32),
            pltpu.VMEM((16, F), jnp.float32),
        ],
    )(hp, msg, rowm, colm, ewm, disA)

    o3_pad = pl.pallas_call(
        _tc_post,
        out_shape=jax.ShapeDtypeStruct((N_PAD, OUT), jnp.float32),
    )(accp, disA, xp, hp, Wc0, Wc1, bc, Wr, ba, Wl, bl)

    return (x, o3_pad[:N])


# speedup vs baseline: 40.5397x; 1.2845x over previous
import jax
import jax.numpy as jnp
from jax import lax
from jax.experimental import pallas as pl
from jax.experimental.pallas import tpu as pltpu
from jax.experimental.pallas import tpu_sc as plsc

N = 10000
E = 320000
HID = 64
OUT = 64
F = 128

NC = 2
NS = 16
NW = NC * NS

EC_W = 80
E_PAD = NW * EC_W * 128
EROWS = E_PAD // 128
EC_T = EROWS // NS
EB_W = 16

NP_T = 640
N_PAD = NS * NP_T


def _sc_mesh():
    return plsc.VectorSubcoreMesh(
        core_axis_name="c", subcore_axis_name="s",
        num_cores=NC, num_subcores=NS)


def _rsqrt_masked(d):
    dn = jnp.maximum(d, jnp.float32(1e-35))
    i = lax.bitcast_convert_type(dn, jnp.int32)
    y = lax.bitcast_convert_type(jnp.int32(0x5F3759DF) - (i >> 1),
                                 jnp.float32)
    for _ in range(3):
        y = y * (jnp.float32(1.5) - jnp.float32(0.5) * dn * y * y)
    return jnp.where(d > jnp.float32(0.0), y, jnp.float32(0.0))


def _sc_degrees(rowm, colm, ewm, disA,
                deg_sh, row_v, col_v, ew_v, zb_v, dloc_v):
    cid = lax.axis_index("c")
    sid = lax.axis_index("s")

    for j in range(NP_T // 16):
        zb_v[pl.ds(16 * j, 16)] = jnp.zeros((16,), jnp.float32)
    pltpu.sync_copy(zb_v, deg_sh.at[pl.ds(sid * NP_T, NP_T)])
    plsc.subcore_barrier()

    base = sid * EC_T
    pltpu.sync_copy(rowm.at[pl.ds(base, EC_T)], row_v)
    pltpu.sync_copy(colm.at[pl.ds(base, EC_T)], col_v)
    pltpu.sync_copy(ewm.at[pl.ds(base, EC_T)], ew_v)

    @pl.when(cid == 0)
    def _():
        def zero_self(c, _):
            for j in range(8):
                sl = pl.ds(16 * j, 16)
                r = row_v[c, sl]
                cc = col_v[c, sl]
                w = ew_v[c, sl]
                ew_v[c, sl] = jnp.where(r == cc, jnp.float32(0.0), w)
            return 0

        lax.fori_loop(0, EC_T, zero_self, 0)

        def scat(c, _):
            pltpu.sync_copy(ew_v.at[c], deg_sh.at[row_v.at[c]], add=True)
            return 0

        lax.fori_loop(0, EC_T, scat, 0)

    @pl.when(cid == 1)
    def _():
        def scat(c, _):
            pltpu.sync_copy(ew_v.at[c], deg_sh.at[col_v.at[c]], add=True)
            return 0

        lax.fori_loop(0, EC_T, scat, 0)

    plsc.subcore_barrier()

    pltpu.sync_copy(deg_sh.at[pl.ds(sid * NP_T, NP_T)], dloc_v)
    for j in range(NP_T // 16):
        sl = pl.ds(16 * j, 16)
        dloc_v[sl] = _rsqrt_masked(dloc_v[sl])
    pltpu.sync_copy(dloc_v, disA.at[cid, pl.ds(sid * NP_T, NP_T)])


def _sc_edge_pass(hp, msg, rowm, colm, ewm, disA, accp, tbl,
                  acc_sh, disl_v, dis2l_v, hblk_v, mblk_v, tblk_v,
                  row_v, col_v, ew_v, g0_v, g1_v, zb_v, sg0, sg1):
    cid = lax.axis_index("c")
    sid = lax.axis_index("s")
    wid = sid * NC + cid
    nbase = sid * NP_T

    pltpu.sync_copy(disA.at[0, pl.ds(nbase, NP_T)], disl_v)
    pltpu.sync_copy(disA.at[1, pl.ds(nbase, NP_T)], dis2l_v)

    for i in range(16):
        for j in range(8):
            zb_v[i, pl.ds(16 * j, 16)] = jnp.zeros((16,), jnp.float32)

    def zacc(jj, _):
        pltpu.sync_copy(zb_v, acc_sh.at[pl.ds(nbase + 16 * jj, 16)])
        return 0

    lax.fori_loop(0, NP_T // 16, zacc, 0)

    def tb(b, _):
        rbase = nbase + 16 * b
        pltpu.sync_copy(hp.at[pl.ds(rbase, 16)], hblk_v)
        pltpu.sync_copy(msg.at[pl.ds(rbase, 16)], mblk_v)
        d16 = disl_v[pl.ds(16 * b, 16)]
        d216 = dis2l_v[pl.ds(16 * b, 16)]
        for k in range(16):
            dk = jnp.full((16,), d16[k], jnp.float32)
            d2k = jnp.full((16,), d216[k], jnp.float32)
            for j in range(4):
                sl = pl.ds(16 * j, 16)
                sl2 = pl.ds(HID + 16 * j, 16)
                tblk_v[k, sl] = hblk_v[k, sl] * dk
                tblk_v[k, sl2] = mblk_v[k, sl] * d2k
        pltpu.sync_copy(tblk_v, tbl.at[pl.ds(rbase, 16)])
        return 0

    lax.fori_loop(0, NP_T // 16, tb, 0)
    plsc.subcore_barrier()

    ebase = wid * EC_W

    def scale_chunk(c, gbuf):
        def egroup(g, _):
            sl16 = pl.ds(16 * g, 16)
            r16 = row_v[c, sl16]
            c16 = col_v[c, sl16]
            w16 = ew_v[c, sl16]
            wz16 = jnp.where(r16 == c16, jnp.float32(0.0), w16)
            for k in range(16):
                e = 16 * g + k
                wzk = jnp.full((16,), wz16[k], jnp.float32)
                wk = jnp.full((16,), w16[k], jnp.float32)
                for j in range(4):
                    sl = pl.ds(16 * j, 16)
                    sl2 = pl.ds(HID + 16 * j, 16)
                    gbuf[e, sl] = gbuf[e, sl] * wzk
                    gbuf[e, sl2] = gbuf[e, sl2] * wk
            return 0

        lax.fori_loop(0, 8, egroup, 0)

    def block(b, _):
        pltpu.sync_copy(rowm.at[pl.ds(ebase + EB_W * b, EB_W)], row_v)
        pltpu.sync_copy(colm.at[pl.ds(ebase + EB_W * b, EB_W)], col_v)
        pltpu.sync_copy(ewm.at[pl.ds(ebase + EB_W * b, EB_W)], ew_v)

        pltpu.async_copy(tbl.at[row_v.at[0]], g0_v, sg0)

        def pair(i, _):
            lc0 = 2 * i
            lc1 = 2 * i + 1
            pltpu.make_async_copy(tbl.at[row_v.at[lc0]], g0_v, sg0).wait()
            pltpu.async_copy(tbl.at[row_v.at[lc1]], g1_v, sg1)
            scale_chunk(lc0, g0_v)
            pltpu.sync_copy(g0_v, acc_sh.at[col_v.at[lc0]], add=True)

            pltpu.make_async_copy(tbl.at[row_v.at[lc1]], g1_v, sg1).wait()

            @pl.when(i < EB_W // 2 - 1)
            def _():
                pltpu.async_copy(tbl.at[row_v.at[lc0 + 2]], g0_v, sg0)

            scale_chunk(lc1, g1_v)
            pltpu.sync_copy(g1_v, acc_sh.at[col_v.at[lc1]], add=True)
            return 0

        lax.fori_loop(0, EB_W // 2, pair, 0)
        return 0

    lax.fori_loop(0, EC_W // EB_W, block, 0)
    plsc.subcore_barrier()

    pltpu.sync_copy(acc_sh.at[pl.ds(nbase, NP_T)],
                    accp.at[cid, pl.ds(nbase, NP_T)])


def _tc_pre(h_ref, x_ref, Wh_ref, bh_ref, Wx_ref, bx_ref, Wi_ref,
            hp_ref, msg_ref, xp_ref):
    h = h_ref[...]
    x = x_ref[...]
    dnT = (((1,), (1,)), ((), ()))
    hp = lax.dot_general(h, Wh_ref[...], dnT,
                         preferred_element_type=jnp.float32) + bh_ref[...][None, :]
    xp = lax.dot_general(x, Wx_ref[...], dnT,
                         preferred_element_type=jnp.float32) + bx_ref[...][None, :]
    msg = lax.dot_general(xp, Wi_ref[...], (((1,), (0,)), ((), ())),
                          preferred_element_type=jnp.float32)
    hp_ref[...] = hp
    msg_ref[...] = msg
    xp_ref[...] = xp


def _tc_post(accp_ref, disA_ref, xp_ref, hp_ref, Wc0_ref, Wc1_ref,
             bc_ref, Wr_ref, ba_ref, Wl_ref, bl_ref, out_ref):
    dis = disA_ref[0]
    dis2 = disA_ref[1]
    acc = accp_ref[0] + accp_ref[1]
    tx1 = (-dis)[:, None] * acc[:, :HID]
    prop = dis2[:, None] * acc[:, HID:]
    dnT = (((1,), (1,)), ((), ()))
    o_cheb = (lax.dot_general(hp_ref[...], Wc0_ref[...], dnT,
                              preferred_element_type=jnp.float32)
              + lax.dot_general(tx1, Wc1_ref[...], dnT,
                                preferred_element_type=jnp.float32)
              + bc_ref[...][None, :])
    o_arma = prop + lax.dot_general(xp_ref[...], Wr_ref[...],
                                    (((1,), (0,)), ((), ())),
                                    preferred_element_type=jnp.float32)
    o_arma = o_arma + ba_ref[...][None, :]
    o_arma = jnp.maximum(o_arma, jnp.float32(0.0))
    slope = jnp.float32(0.01)
    o1 = jnp.where(o_cheb >= 0, o_cheb, slope * o_cheb)
    o2 = jnp.where(o_arma >= 0, o_arma, slope * o_arma)
    out_ref[...] = (lax.dot_general(o1 + o2, Wl_ref[...], dnT,
                                    preferred_element_type=jnp.float32)
                    + bl_ref[...][None, :])


def kernel(h, x, edge_weight, Wh, bh, Wx, bx, Wc0, Wc1, bc, Wi, Wr, ba, Wl,
           bl, edge_index):
    row = edge_index[0].astype(jnp.int32)
    col = edge_index[1].astype(jnp.int32)
    ew = edge_weight.astype(jnp.float32)

    npad = E_PAD - E
    pad_idx = jnp.arange(npad, dtype=jnp.int32) % N
    rowm = jnp.concatenate([row, pad_idx]).reshape(EROWS, 128)
    colm = jnp.concatenate([col, pad_idx]).reshape(EROWS, 128)
    ewm = jnp.concatenate([ew, jnp.zeros((npad,), jnp.float32)]
                          ).reshape(EROWS, 128)

    h_pad = jnp.zeros((N_PAD, h.shape[1]), jnp.float32).at[:N].set(h)
    x_pad = jnp.zeros((N_PAD, x.shape[1]), jnp.float32).at[:N].set(x)

    hp, msg, xp = pl.pallas_call(
        _tc_pre,
        out_shape=(
            jax.ShapeDtypeStruct((N_PAD, HID), jnp.float32),
            jax.ShapeDtypeStruct((N_PAD, HID), jnp.float32),
            jax.ShapeDtypeStruct((N_PAD, HID), jnp.float32),
        ),
    )(h_pad, x_pad, Wh, bh, Wx, bx, Wi)

    disA = pl.kernel(
        _sc_degrees,
        out_type=jax.ShapeDtypeStruct((NC, N_PAD), jnp.float32),
        mesh=_sc_mesh(),
        compiler_params=pltpu.CompilerParams(needs_layout_passes=False),
        scratch_types=[
            pltpu.VMEM_SHARED((N_PAD,), jnp.float32),
            pltpu.VMEM((EC_T, 128), jnp.int32),
            pltpu.VMEM((EC_T, 128), jnp.int32),
            pltpu.VMEM((EC_T, 128), jnp.float32),
            pltpu.VMEM((NP_T,), jnp.float32),
            pltpu.VMEM((NP_T,), jnp.float32),
        ],
    )(rowm, colm, ewm)

    accp, _tbl = pl.kernel(
        _sc_edge_pass,
        out_type=(
            jax.ShapeDtypeStruct((NC, N_PAD, F), jnp.float32),
            jax.ShapeDtypeStruct((N_PAD, F), jnp.float32),
        ),
        mesh=_sc_mesh(),
        compiler_params=pltpu.CompilerParams(needs_layout_passes=False),
        scratch_types=[
            pltpu.VMEM_SHARED((N_PAD, F), jnp.float32),
            pltpu.VMEM((NP_T,), jnp.float32),
            pltpu.VMEM((NP_T,), jnp.float32),
            pltpu.VMEM((16, HID), jnp.float32),
            pltpu.VMEM((16, HID), jnp.float32),
            pltpu.VMEM((16, F), jnp.float32),
            pltpu.VMEM((EB_W, 128), jnp.int32),
            pltpu.VMEM((EB_W, 128), jnp.int32),
            pltpu.VMEM((EB_W, 128), jnp.float32),
            pltpu.VMEM((128, F), jnp.float32),
            pltpu.VMEM((128, F), jnp.float32),
            pltpu.VMEM((16, F), jnp.float32),
            pltpu.SemaphoreType.DMA,
            pltpu.SemaphoreType.DMA,
        ],
    )(hp, msg, rowm, colm, ewm, disA)

    o3_pad = pl.pallas_call(
        _tc_post,
        out_shape=jax.ShapeDtypeStruct((N_PAD, OUT), jnp.float32),
    )(accp, disA, xp, hp, Wc0, Wc1, bc, Wr, ba, Wl, bl)

    return (x, o3_pad[:N])
